# baseline (speedup 1.0000x reference)
import jax
import jax.numpy as jnp
from jax import lax
from jax.experimental import pallas as pl
from jax.experimental.pallas import tpu as pltpu
from jax.experimental.pallas import tpu_sc as plsc

N_USERS = 25000
N_ITEMS = 25000
N = N_USERS + N_ITEMS
E = 800000
DIM = 64
HALF = 16
NQ = DIM // HALF

NC = 2
NS = 16
LANES = 16

N_PAD = 51200
TN = N_PAD // NS
NCHUNK = 128
N_NCH = TN // NCHUNK

ECHUNK = 512
SUP = 4
E_TILE = 51200
CH = 100
N_SUP = CH // SUP
E_PAD = E_TILE * NS

_MESH = plsc.VectorSubcoreMesh(core_axis_name="c", subcore_axis_name="s")
_PARAMS = pltpu.CompilerParams(needs_layout_passes=False,
                               use_tc_tiling_on_sc=False)


def _rsqrt16(d):
    i = plsc.bitcast(d, jnp.int32)
    i = jnp.int32(0x5F3759DF) - (i >> 1)
    y = plsc.bitcast(i, jnp.float32)
    for _ in range(3):
        y = y * (jnp.float32(1.5) - jnp.float32(0.5) * d * y * y)
    return jnp.where(d > 0.0, y, jnp.float32(0.0))


def _edge_stream(idx_t, table, acc_sh, ibuf, gbuf, w, isem, gsems, ssems):
    def gath(b, k, slot):
        return pltpu.async_copy(table.at[ibuf.at[b, 0, k]], gbuf.at[slot],
                                gsems[slot])

    def gwait(b, k, slot):
        pltpu.make_async_copy(table.at[ibuf.at[b, 0, k]], gbuf.at[slot],
                              gsems[slot]).wait()

    def scat(b, k, slot):
        return pltpu.async_copy(gbuf.at[slot], acc_sh.at[ibuf.at[b, 1, k]],
                                ssems[slot], add=True)

    def swait(b, k, slot):
        pltpu.make_async_copy(gbuf.at[slot], acc_sh.at[ibuf.at[b, 1, k]],
                              ssems[slot]).wait()

    pltpu.sync_copy(idx_t.at[w, 0], ibuf.at[0])

    @pl.loop(0, N_SUP)
    def _(jj):
        b = jj % 2

        @pl.when(jj > 0)
        def _():
            pltpu.make_async_copy(idx_t.at[w, jj], ibuf.at[b], isem).wait()

        for k in range(SUP):
            slot = k

            @pl.when(jj > 0)
            def _():
                swait(b, k, slot)

            gath(b, k, slot)
            if k > 0:
                gwait(b, k - 1, k - 1)
                scat(b, k - 1, k - 1)
            else:
                @pl.when(jj > 0)
                def _():
                    gwait(b, SUP - 1, SUP - 1)
                    scat(1 - b, SUP - 1, SUP - 1)

            if k == SUP - 1:
                @pl.when(jj + 1 < N_SUP)
                def _():
                    pltpu.async_copy(idx_t.at[w, jj + 1], ibuf.at[1 - b],
                                     isem)

    gwait(0, SUP - 1, SUP - 1)
    scat(0, SUP - 1, SUP - 1)
    for slot in range(SUP):
        swait(0, slot, slot)


def _scale_rows(dvbuf, i, src_buf, dst_buf, extra=None):
    lsl = pl.ds(0, LANES)
    for m in range(NCHUNK // LANES):
        dvec = dvbuf[pl.ds(i * NCHUNK + m * LANES, LANES)]
        for n2 in range(LANES):
            n = m * LANES + n2
            dv = dvec[n2]
            v = src_buf[n, lsl] * dv
            dst_buf[n, lsl] = v
            if extra is not None:
                extra[n, lsl] = v * dv


def _mega_body(idx_t, xq0, xq1, xq2, xq3,
               oq0, oq1, oq2, oq3,
               deg_sh, acc_sh, x1h0, x1h1, x1h2, x1h3, x2h, y1h,
               ibuf, gbuf, dvbuf, zrow, onesb, sbuf, obuf, obuf2,
               isem, gsem0, gsem1, gsem2, gsem3, ssem0, ssem1, ssem2, ssem3):
    gsems = (gsem0, gsem1, gsem2, gsem3)
    ssems = (ssem0, ssem1, ssem2, ssem3)
    c = lax.axis_index("c")
    s = lax.axis_index("s")
    w = c * NS + s
    nbase = (c * N_PAD + s * TN).astype(jnp.int32)
    abase = s * TN
    z = jnp.zeros((LANES,), jnp.float32)
    one = jnp.full((LANES,), 1.0, jnp.float32)
    third = jnp.float32(1.0 / 3.0)
    lsl = pl.ds(0, LANES)
    xqs = (xq0, xq1, xq2, xq3)
    oqs = (oq0, oq1, oq2, oq3)
    x1hs = (x1h0, x1h1, x1h2, x1h3)

    @pl.loop(0, ECHUNK // LANES)
    def _(i):
        onesb[pl.ds(i * LANES, LANES)] = one

    @pl.loop(0, TN // LANES)
    def _(i):
        dvbuf[pl.ds(i * LANES, LANES)] = z

    @pl.loop(0, NCHUNK)
    def _(n):
        zrow[n, lsl] = z

    pltpu.sync_copy(dvbuf, deg_sh.at[pl.ds(abase, TN)])

    @pl.loop(0, N_NCH)
    def _(i):
        pltpu.sync_copy(zrow, acc_sh.at[pl.ds(abase + i * NCHUNK, NCHUNK)])

    plsc.subcore_barrier()

    def d_scat(b, k, slot):
        return pltpu.async_copy(onesb, deg_sh.at[ibuf.at[b, 1, k]],
                                ssems[slot], add=True)

    def d_swait(b, k, slot):
        pltpu.make_async_copy(onesb, deg_sh.at[ibuf.at[b, 1, k]],
                              ssems[slot]).wait()

    pltpu.sync_copy(idx_t.at[w, 0], ibuf.at[0])

    @pl.loop(0, N_SUP)
    def _(jj):
        b = jj % 2

        @pl.when(jj > 0)
        def _():
            pltpu.make_async_copy(idx_t.at[w, jj], ibuf.at[b], isem).wait()

        for k in range(SUP):
            slot = k

            @pl.when(jj > 0)
            def _():
                d_swait(b, k, slot)

            d_scat(b, k, slot)
            if k == SUP - 1:
                @pl.when(jj + 1 < N_SUP)
                def _():
                    pltpu.async_copy(idx_t.at[w, jj + 1], ibuf.at[1 - b],
                                     isem)

    for slot in range(SUP):
        d_swait(0, slot, slot)

    plsc.subcore_barrier()

    pltpu.sync_copy(deg_sh.at[pl.ds(abase, TN)], dvbuf)

    @pl.loop(0, TN // LANES)
    def _(i):
        sl = pl.ds(i * LANES, LANES)
        dvbuf[sl] = _rsqrt16(dvbuf[sl])

    for q in range(NQ):
        @pl.loop(0, N_NCH)
        def _(i):
            base = nbase + i * NCHUNK
            pltpu.sync_copy(xqs[q].at[pl.ds(base, NCHUNK)], sbuf)
            _scale_rows(dvbuf, i, sbuf, obuf)
            pltpu.sync_copy(obuf, x1hs[q].at[pl.ds(base, NCHUNK)])

    plsc.subcore_barrier()

    for q in range(NQ):
        _edge_stream(idx_t, x1hs[q], acc_sh, ibuf, gbuf, w, isem, gsems,
                     ssems)
        plsc.subcore_barrier()

        @pl.loop(0, N_NCH)
        def _(i):
            base = nbase + i * NCHUNK
            asl = pl.ds(abase + i * NCHUNK, NCHUNK)
            pltpu.sync_copy(acc_sh.at[asl], sbuf)
            pltpu.sync_copy(zrow, acc_sh.at[asl])
            _scale_rows(dvbuf, i, sbuf, obuf, extra=obuf2)
            pltpu.sync_copy(obuf, y1h.at[pl.ds(base, NCHUNK)])
            pltpu.sync_copy(obuf2, x2h.at[pl.ds(base, NCHUNK)])

        plsc.subcore_barrier()

        _edge_stream(idx_t, x2h, acc_sh, ibuf, gbuf, w, isem, gsems, ssems)
        plsc.subcore_barrier()

        @pl.loop(0, N_NCH)
        def _(i):
            base = nbase + i * NCHUNK
            asl = pl.ds(abase + i * NCHUNK, NCHUNK)
            pltpu.sync_copy(acc_sh.at[asl], sbuf)
            pltpu.sync_copy(zrow, acc_sh.at[asl])
            pltpu.sync_copy(xqs[q].at[pl.ds(base, NCHUNK)], obuf)
            pltpu.sync_copy(y1h.at[pl.ds(base, NCHUNK)], obuf2)
            for m in range(NCHUNK // LANES):
                dvec = dvbuf[pl.ds(i * NCHUNK + m * LANES, LANES)]
                for n2 in range(LANES):
                    n = m * LANES + n2
                    dv = dvec[n2]
                    o = obuf[n, lsl] + obuf2[n, lsl] + sbuf[n, lsl] * dv
                    obuf[n, lsl] = o * third
            pltpu.sync_copy(obuf, oqs[q].at[pl.ds(base, NCHUNK)])

        plsc.subcore_barrier()


def _f32(shape):
    return jax.ShapeDtypeStruct(shape, jnp.float32)


_mega = pl.kernel(
    _mega_body,
    out_type=tuple([_f32((NC * N_PAD, HALF))] * NQ),
    mesh=_MESH,
    compiler_params=_PARAMS,
    scratch_types=[
        pltpu.VMEM_SHARED((N_PAD,), jnp.float32),
        pltpu.VMEM_SHARED((N_PAD, HALF), jnp.float32),
        pltpu.HBM((NC * N_PAD, HALF), jnp.float32),
        pltpu.HBM((NC * N_PAD, HALF), jnp.float32),
        pltpu.HBM((NC * N_PAD, HALF), jnp.float32),
        pltpu.HBM((NC * N_PAD, HALF), jnp.float32),
        pltpu.HBM((NC * N_PAD, HALF), jnp.float32),
        pltpu.HBM((NC * N_PAD, HALF), jnp.float32),
        pltpu.VMEM((2, 2, SUP, ECHUNK), jnp.int32),
        pltpu.VMEM((SUP, ECHUNK, HALF), jnp.float32),
        pltpu.VMEM((TN,), jnp.float32),
        pltpu.VMEM((NCHUNK, HALF), jnp.float32),
        pltpu.VMEM((ECHUNK,), jnp.float32),
        pltpu.VMEM((NCHUNK, HALF), jnp.float32),
        pltpu.VMEM((NCHUNK, HALF), jnp.float32),
        pltpu.VMEM((NCHUNK, HALF), jnp.float32),
    ] + [pltpu.SemaphoreType.DMA] * 9,
)


def kernel(pos_edge_index, neg_edge_index, user_pos_embedding,
           item_pos_embedding, user_neg_embedding, item_neg_embedding):
    x = jnp.stack([
        jnp.concatenate([user_pos_embedding, item_pos_embedding], axis=0),
        jnp.concatenate([user_neg_embedding, item_neg_embedding], axis=0),
    ])
    x = jnp.pad(x, ((0, 0), (0, N_PAD - N), (0, 0)))
    xq = [x[:, :, q * HALF:(q + 1) * HALF].reshape(NC * N_PAD, HALF)
          for q in range(NQ)]

    rows = jnp.stack([pos_edge_index[0], neg_edge_index[0]])
    cols = jnp.stack([pos_edge_index[1], neg_edge_index[1]])
    pad = E_PAD - E
    rows = jnp.pad(rows, ((0, 0), (0, pad)))
    cols = jnp.pad(cols, ((0, 0), (0, pad)), constant_values=N)
    rows = rows + (jnp.arange(NC, dtype=jnp.int32) * N_PAD)[:, None]
    rows_t = rows.reshape(NC * NS, N_SUP, SUP, ECHUNK)
    cols_t = cols.reshape(NC * NS, N_SUP, SUP, ECHUNK)
    idx_t = jnp.stack([rows_t, cols_t], axis=2)

    outq = _mega(idx_t, *xq)

    out = jnp.concatenate([o.reshape(NC, N_PAD, 1, HALF) for o in outq],
                          axis=2).reshape(NC, N_PAD, DIM)
    return (out[0, :N_USERS], out[0, N_USERS:N],
            out[1, :N_USERS], out[1, N_USERS:N])


# speedup vs baseline: 1.3812x; 1.3812x over previous
import jax
import jax.numpy as jnp
from jax import lax
from jax.experimental import pallas as pl
from jax.experimental.pallas import tpu as pltpu
from jax.experimental.pallas import tpu_sc as plsc

N_USERS = 25000
N_ITEMS = 25000
N = N_USERS + N_ITEMS
E = 800000
DIM = 64
HALF = 16
NQ = DIM // HALF

NC = 2
NS = 16
LANES = 16

N_PAD = 51200
TN = N_PAD // NS
NCHUNK = 128
N_NCH = TN // NCHUNK

ECHUNK = 512
SUP = 2
E_TILE = 50176
CH = 98
N_SUP = CH // SUP
E_PAD = E_TILE * NS

_MESH = plsc.VectorSubcoreMesh(core_axis_name="c", subcore_axis_name="s")
_PARAMS = pltpu.CompilerParams(needs_layout_passes=False,
                               use_tc_tiling_on_sc=False)


def _rsqrt16(d):
    i = plsc.bitcast(d, jnp.int32)
    i = jnp.int32(0x5F3759DF) - (i >> 1)
    y = plsc.bitcast(i, jnp.float32)
    for _ in range(3):
        y = y * (jnp.float32(1.5) - jnp.float32(0.5) * d * y * y)
    return jnp.where(d > 0.0, y, jnp.float32(0.0))


def _edge_stream(idx_t, table, acc_sh, ibuf, gbuf, w, isem, gsems, ssems):
    def gath(b, k, slot):
        return pltpu.async_copy(table.at[ibuf.at[b, 0, k]], gbuf.at[slot],
                                gsems[slot])

    def gwait(b, k, slot):
        pltpu.make_async_copy(table.at[ibuf.at[b, 0, k]], gbuf.at[slot],
                              gsems[slot]).wait()

    def scat(b, k, slot):
        return pltpu.async_copy(gbuf.at[slot], acc_sh.at[ibuf.at[b, 1, k]],
                                ssems[slot], add=True)

    def swait(b, k, slot):
        pltpu.make_async_copy(gbuf.at[slot], acc_sh.at[ibuf.at[b, 1, k]],
                              ssems[slot]).wait()

    pltpu.sync_copy(idx_t.at[w, 0], ibuf.at[0])

    @pl.loop(0, N_SUP)
    def _(jj):
        b = jj % 2

        @pl.when(jj > 0)
        def _():
            pltpu.make_async_copy(idx_t.at[w, jj], ibuf.at[b], isem).wait()

        for k in range(SUP):
            slot = k

            @pl.when(jj > 0)
            def _():
                swait(b, k, slot)

            gath(b, k, slot)
            if k > 0:
                gwait(b, k - 1, k - 1)
                scat(b, k - 1, k - 1)
            else:
                @pl.when(jj > 0)
                def _():
                    gwait(b, SUP - 1, SUP - 1)
                    scat(1 - b, SUP - 1, SUP - 1)

            if k == SUP - 1:
                @pl.when(jj + 1 < N_SUP)
                def _():
                    pltpu.async_copy(idx_t.at[w, jj + 1], ibuf.at[1 - b],
                                     isem)

    gwait(0, SUP - 1, SUP - 1)
    scat(0, SUP - 1, SUP - 1)
    for slot in range(SUP):
        swait(0, slot, slot)


def _flush_pipeline(issue_loads, wait_load, issue_stores, wait_store,
                    n_loads, n_stores, compute):
    issue_loads(0, 0)

    @pl.loop(0, N_NCH)
    def _(i):
        b = i % 2
        for _ in range(n_loads):
            wait_load()

        @pl.when(i + 1 < N_NCH)
        def _():
            issue_loads(i + 1, 1 - b)

        @pl.when(i >= 2)
        def _():
            for _ in range(n_stores):
                wait_store()

        compute(i, b)
        issue_stores(i, b)

    for _ in range(2 * n_stores):
        wait_store()


def _mega_body(idx_t, xq0, xq1, xq2, xq3,
               oq0, oq1, oq2, oq3,
               deg_sh, acc_sh, x1h0, x1h1, x1h2, x1h3, x2h, y1h,
               ibuf, gbuf, dvbuf, zrow, onesb,
               sbufs, abufs, bbufs, obufs, o2bufs,
               isem, gsem0, gsem1, gsem2, gsem3, ssem0, ssem1, ssem2, ssem3,
               lsem, osem):
    gsems = (gsem0, gsem1, gsem2, gsem3)
    ssems = (ssem0, ssem1, ssem2, ssem3)
    c = lax.axis_index("c")
    s = lax.axis_index("s")
    w = c * NS + s
    nbase = (c * N_PAD + s * TN).astype(jnp.int32)
    abase = s * TN
    z = jnp.zeros((LANES,), jnp.float32)
    one = jnp.full((LANES,), 1.0, jnp.float32)
    third = jnp.float32(1.0 / 3.0)
    lsl = pl.ds(0, LANES)
    xqs = (xq0, xq1, xq2, xq3)
    oqs = (oq0, oq1, oq2, oq3)
    x1hs = (x1h0, x1h1, x1h2, x1h3)

    @pl.loop(0, ECHUNK // LANES)
    def _(i):
        onesb[pl.ds(i * LANES, LANES)] = one

    @pl.loop(0, TN // LANES)
    def _(i):
        dvbuf[pl.ds(i * LANES, LANES)] = z

    @pl.loop(0, NCHUNK)
    def _(n):
        zrow[n, lsl] = z

    pltpu.sync_copy(dvbuf, deg_sh.at[pl.ds(abase, TN)])

    @pl.loop(0, N_NCH)
    def _(i):
        pltpu.sync_copy(zrow, acc_sh.at[pl.ds(abase + i * NCHUNK, NCHUNK)])

    plsc.subcore_barrier()

    def d_scat(b, k, slot):
        return pltpu.async_copy(onesb, deg_sh.at[ibuf.at[b, 1, k]],
                                ssems[slot], add=True)

    def d_swait(b, k, slot):
        pltpu.make_async_copy(onesb, deg_sh.at[ibuf.at[b, 1, k]],
                              ssems[slot]).wait()

    pltpu.sync_copy(idx_t.at[w, 0], ibuf.at[0])

    @pl.loop(0, N_SUP)
    def _(jj):
        b = jj % 2

        @pl.when(jj > 0)
        def _():
            pltpu.make_async_copy(idx_t.at[w, jj], ibuf.at[b], isem).wait()

        for k in range(SUP):
            slot = k

            @pl.when(jj > 0)
            def _():
                d_swait(b, k, slot)

            d_scat(b, k, slot)
            if k == SUP - 1:
                @pl.when(jj + 1 < N_SUP)
                def _():
                    pltpu.async_copy(idx_t.at[w, jj + 1], ibuf.at[1 - b],
                                     isem)

    for slot in range(SUP):
        d_swait(0, slot, slot)

    plsc.subcore_barrier()

    pltpu.sync_copy(deg_sh.at[pl.ds(abase, TN)], dvbuf)

    @pl.loop(0, TN // LANES)
    def _(i):
        sl = pl.ds(i * LANES, LANES)
        dvbuf[sl] = _rsqrt16(dvbuf[sl])

    def lwait():
        pltpu.make_async_copy(acc_sh.at[pl.ds(abase, NCHUNK)], sbufs.at[0],
                              lsem).wait()

    def swaitf():
        pltpu.make_async_copy(obufs.at[0], y1h.at[pl.ds(nbase, NCHUNK)],
                              osem).wait()

    for q in range(NQ):
        def p3_loads(i, b, q=q):
            pltpu.async_copy(xqs[q].at[pl.ds(nbase + i * NCHUNK, NCHUNK)],
                             abufs.at[b], lsem)

        def p3_compute(i, b):
            for m in range(NCHUNK // LANES):
                dvec = dvbuf[pl.ds(i * NCHUNK + m * LANES, LANES)]
                for n2 in range(LANES):
                    n = m * LANES + n2
                    dv = dvec[n2]
                    obufs[b, n, lsl] = abufs[b, n, lsl] * dv

        def p3_stores(i, b, q=q):
            pltpu.async_copy(obufs.at[b],
                             x1hs[q].at[pl.ds(nbase + i * NCHUNK, NCHUNK)],
                             osem)

        _flush_pipeline(p3_loads, lwait, p3_stores, swaitf, 1, 1, p3_compute)

    plsc.subcore_barrier()

    for q in range(NQ):
        _edge_stream(idx_t, x1hs[q], acc_sh, ibuf, gbuf, w, isem, gsems,
                     ssems)
        plsc.subcore_barrier()

        @pl.loop(0, N_NCH)
        def _(i):
            base = nbase + i * NCHUNK
            asl = pl.ds(abase + i * NCHUNK, NCHUNK)
            pltpu.sync_copy(acc_sh.at[asl], sbufs.at[0])
            pltpu.sync_copy(zrow, acc_sh.at[asl])
            for m in range(NCHUNK // LANES):
                dvec = dvbuf[pl.ds(i * NCHUNK + m * LANES, LANES)]
                for n2 in range(LANES):
                    n = m * LANES + n2
                    dv = dvec[n2]
                    v = sbufs[0, n, lsl] * dv
                    obufs[0, n, lsl] = v
                    o2bufs[0, n, lsl] = v * dv
            pltpu.sync_copy(obufs.at[0], y1h.at[pl.ds(base, NCHUNK)])
            pltpu.sync_copy(o2bufs.at[0], x2h.at[pl.ds(base, NCHUNK)])

        plsc.subcore_barrier()

        _edge_stream(idx_t, x2h, acc_sh, ibuf, gbuf, w, isem, gsems, ssems)
        plsc.subcore_barrier()

        @pl.loop(0, N_NCH)
        def _(i):
            base = nbase + i * NCHUNK
            asl = pl.ds(abase + i * NCHUNK, NCHUNK)
            pltpu.sync_copy(acc_sh.at[asl], sbufs.at[0])
            pltpu.sync_copy(zrow, acc_sh.at[asl])
            pltpu.sync_copy(xqs[q].at[pl.ds(base, NCHUNK)], abufs.at[0])
            pltpu.sync_copy(y1h.at[pl.ds(base, NCHUNK)], bbufs.at[0])
            for m in range(NCHUNK // LANES):
                dvec = dvbuf[pl.ds(i * NCHUNK + m * LANES, LANES)]
                for n2 in range(LANES):
                    n = m * LANES + n2
                    dv = dvec[n2]
                    o = (abufs[0, n, lsl] + bbufs[0, n, lsl]
                         + sbufs[0, n, lsl] * dv)
                    obufs[0, n, lsl] = o * third
            pltpu.sync_copy(obufs.at[0], oqs[q].at[pl.ds(base, NCHUNK)])

        plsc.subcore_barrier()


def _f32(shape):
    return jax.ShapeDtypeStruct(shape, jnp.float32)


_mega = pl.kernel(
    _mega_body,
    out_type=tuple([_f32((NC * N_PAD, HALF))] * NQ),
    mesh=_MESH,
    compiler_params=_PARAMS,
    scratch_types=[
        pltpu.VMEM_SHARED((N_PAD,), jnp.float32),
        pltpu.VMEM_SHARED((N_PAD, HALF), jnp.float32),
        pltpu.HBM((NC * N_PAD, HALF), jnp.float32),
        pltpu.HBM((NC * N_PAD, HALF), jnp.float32),
        pltpu.HBM((NC * N_PAD, HALF), jnp.float32),
        pltpu.HBM((NC * N_PAD, HALF), jnp.float32),
        pltpu.HBM((NC * N_PAD, HALF), jnp.float32),
        pltpu.HBM((NC * N_PAD, HALF), jnp.float32),
        pltpu.VMEM((2, 2, SUP, ECHUNK), jnp.int32),
        pltpu.VMEM((SUP, ECHUNK, HALF), jnp.float32),
        pltpu.VMEM((TN,), jnp.float32),
        pltpu.VMEM((NCHUNK, HALF), jnp.float32),
        pltpu.VMEM((ECHUNK,), jnp.float32),
        pltpu.VMEM((2, NCHUNK, HALF), jnp.float32),
        pltpu.VMEM((2, NCHUNK, HALF), jnp.float32),
        pltpu.VMEM((2, NCHUNK, HALF), jnp.float32),
        pltpu.VMEM((2, NCHUNK, HALF), jnp.float32),
        pltpu.VMEM((2, NCHUNK, HALF), jnp.float32),
    ] + [pltpu.SemaphoreType.DMA] * 11,
)


def kernel(pos_edge_index, neg_edge_index, user_pos_embedding,
           item_pos_embedding, user_neg_embedding, item_neg_embedding):
    x = jnp.stack([
        jnp.concatenate([user_pos_embedding, item_pos_embedding], axis=0),
        jnp.concatenate([user_neg_embedding, item_neg_embedding], axis=0),
    ])
    x = jnp.pad(x, ((0, 0), (0, N_PAD - N), (0, 0)))
    xq = [x[:, :, q * HALF:(q + 1) * HALF].reshape(NC * N_PAD, HALF)
          for q in range(NQ)]

    rows = jnp.stack([pos_edge_index[0], neg_edge_index[0]])
    cols = jnp.stack([pos_edge_index[1], neg_edge_index[1]])
    pad = E_PAD - E
    rows = jnp.pad(rows, ((0, 0), (0, pad)))
    cols = jnp.pad(cols, ((0, 0), (0, pad)), constant_values=N)
    rows = rows + (jnp.arange(NC, dtype=jnp.int32) * N_PAD)[:, None]
    rows_t = rows.reshape(NC * NS, N_SUP, SUP, ECHUNK)
    cols_t = cols.reshape(NC * NS, N_SUP, SUP, ECHUNK)
    idx_t = jnp.stack([rows_t, cols_t], axis=2)

    outq = _mega(idx_t, *xq)

    out = jnp.concatenate([o.reshape(NC, N_PAD, 1, HALF) for o in outq],
                          axis=2).reshape(NC, N_PAD, DIM)
    return (out[0, :N_USERS], out[0, N_USERS:N],
            out[1, :N_USERS], out[1, N_USERS:N])
